# Initial kernel scaffold; baseline (speedup 1.0000x reference)
import jax
import jax.numpy as jnp
from jax.experimental import pallas as pl


def kernel(x, lengths, table, W1, b1, W2, b2):
    raise NotImplementedError("write your pallas kernel here")


# speedup vs baseline: 7.7811x; 7.7811x over previous
import functools

import jax
import jax.numpy as jnp
from jax import lax
from jax.experimental import pallas as pl
from jax.experimental.pallas import tpu as pltpu
from jax.experimental.pallas import tpu_sc as plsc

_NC = 2
_NS = 16
_NW = _NC * _NS


def _make_pool(B, L, D):
    BPW = B // _NW
    G = 4
    NGRP = BPW // G
    NCH = D // 16

    mesh = plsc.VectorSubcoreMesh(core_axis_name="c", subcore_axis_name="s")

    @functools.partial(
        pl.kernel,
        mesh=mesh,
        out_type=jax.ShapeDtypeStruct((B, D), jnp.float32),
        scratch_types=[
            pltpu.VMEM((BPW, L), jnp.int32),
            pltpu.VMEM((G * L, D), jnp.float32),
            pltpu.VMEM((BPW, D), jnp.float32),
            pltpu.SemaphoreType.DMA,
        ],
    )
    def pool(x_hbm, table_hbm, out_hbm, idx_v, buf_v, out_v, sem):
        wid = lax.axis_index("s") * _NC + lax.axis_index("c")
        base = wid * BPW
        pltpu.sync_copy(x_hbm.at[pl.ds(base, BPW)], idx_v)

        def accum_row(buf_row0, out_row):
            def tok(t, accs):
                r0 = buf_row0 + 2 * t
                return tuple(
                    accs[d]
                    + buf_v[r0, pl.ds(d * 16, 16)]
                    + buf_v[r0 + 1, pl.ds(d * 16, 16)]
                    for d in range(NCH)
                )
            accs = tuple(jnp.zeros((16,), jnp.float32) for _ in range(NCH))
            accs = lax.fori_loop(0, L // 2, tok, accs)
            for d in range(NCH):
                out_v[out_row, pl.ds(d * 16, 16)] = accs[d]

        def group(g, carry):
            copies = [
                pltpu.async_copy(
                    table_hbm.at[idx_v.at[g * G + r]],
                    buf_v.at[pl.ds(r * L, L)],
                    sem,
                )
                for r in range(G)
            ]
            for c in copies:
                c.wait()
            for r in range(G):
                accum_row(r * L, g * G + r)
            return carry

        lax.fori_loop(0, NGRP, group, 0)
        pltpu.sync_copy(out_v, out_hbm.at[pl.ds(base, BPW)])

    return pool


def _mlp_body(p_ref, len_ref, w1_ref, b1_ref, w2_ref, b2_ref, o_ref):
    rep = p_ref[...] / len_ref[...]
    h = jnp.dot(rep, w1_ref[...], preferred_element_type=jnp.float32)
    h = jnp.maximum(h + b1_ref[...], 0.0)
    o_ref[...] = (
        jnp.dot(h, w2_ref[...], preferred_element_type=jnp.float32)
        + b2_ref[...]
    )


def _mlp(pooled, lens_col, W1, b1r, W2, b2r):
    B, D = pooled.shape
    LATENT = W1.shape[1]
    OUT = W2.shape[1]
    BK = 512
    return pl.pallas_call(
        _mlp_body,
        grid=(B // BK,),
        in_specs=[
            pl.BlockSpec((BK, D), lambda i: (i, 0)),
            pl.BlockSpec((BK, 1), lambda i: (i, 0)),
            pl.BlockSpec((D, LATENT), lambda i: (0, 0)),
            pl.BlockSpec((1, LATENT), lambda i: (0, 0)),
            pl.BlockSpec((LATENT, OUT), lambda i: (0, 0)),
            pl.BlockSpec((1, OUT), lambda i: (0, 0)),
        ],
        out_specs=pl.BlockSpec((BK, OUT), lambda i: (i, 0)),
        out_shape=jax.ShapeDtypeStruct((B, OUT), jnp.float32),
    )(pooled, lens_col, W1, b1r, W2, b2r)


def kernel(x, lengths, table, W1, b1, W2, b2):
    B, L = x.shape
    _, D = table.shape
    pooled = _make_pool(B, L, D)(x.astype(jnp.int32), table)
    lens_col = lengths.astype(jnp.float32).reshape(B, 1)
    return _mlp(
        pooled,
        lens_col,
        W1,
        b1.reshape(1, -1),
        W2,
        b2.reshape(1, -1),
    )
